# baseline (device time: 75514 ns/iter reference)
import jax
import jax.numpy as jnp
from jax import lax
from jax.experimental import pallas as pl
from jax.experimental.pallas import tpu as pltpu

N_DEV = 4
EPS = 1e-5
BM = 1024


def _sumsq_body(x_ref, out_ref):
    x = x_ref[...]
    out_ref[...] = jnp.sum(x * x, axis=1, keepdims=True)


def _allreduce_body(global_n, p_ref, inv_ref, comm_ref, send_sems, recv_sems):
    my = lax.axis_index("i")
    left = lax.rem(my + N_DEV - 1, N_DEV)
    right = lax.rem(my + 1, N_DEV)

    barrier = pltpu.get_barrier_semaphore()
    for nbr in (left, right):
        pl.semaphore_signal(
            barrier, inc=1, device_id=(nbr,),
            device_id_type=pl.DeviceIdType.MESH,
        )
    pl.semaphore_wait(barrier, 2)

    comm_ref[0, :, :] = p_ref[...]

    for h in range(N_DEV - 1):
        rdma = pltpu.make_async_remote_copy(
            src_ref=comm_ref.at[h],
            dst_ref=comm_ref.at[h + 1],
            send_sem=send_sems.at[h],
            recv_sem=recv_sems.at[h],
            device_id=(right,),
            device_id_type=pl.DeviceIdType.MESH,
        )
        rdma.start()
        rdma.wait()

    total = (comm_ref[0, :, :] + comm_ref[1, :, :]
             + comm_ref[2, :, :] + comm_ref[3, :, :])
    inv_ref[...] = lax.rsqrt(total * (1.0 / global_n) + EPS)


def _normalize_body(x_ref, inv_ref, g_ref, out_ref):
    out_ref[...] = x_ref[...] * inv_ref[...] * g_ref[...]


def kernel(x, gamma):
    m, n_loc = x.shape
    global_n = n_loc * N_DEV
    grid = m // BM

    partial = pl.pallas_call(
        _sumsq_body,
        grid=(grid,),
        in_specs=[pl.BlockSpec((BM, n_loc), lambda i: (i, 0))],
        out_specs=pl.BlockSpec((BM, 1), lambda i: (i, 0)),
        out_shape=jax.ShapeDtypeStruct((m, 1), jnp.float32),
    )(x)

    p2 = partial.reshape(m // 128, 128)

    inv2 = pl.pallas_call(
        lambda *refs: _allreduce_body(global_n, *refs),
        out_shape=jax.ShapeDtypeStruct((m // 128, 128), jnp.float32),
        in_specs=[pl.BlockSpec(memory_space=pltpu.VMEM)],
        out_specs=pl.BlockSpec(memory_space=pltpu.VMEM),
        scratch_shapes=[
            pltpu.VMEM((N_DEV, m // 128, 128), jnp.float32),
            pltpu.SemaphoreType.DMA((N_DEV - 1,)),
            pltpu.SemaphoreType.DMA((N_DEV - 1,)),
        ],
        compiler_params=pltpu.CompilerParams(collective_id=0),
    )(p2)

    inv = inv2.reshape(m, 1)
    g2 = gamma.reshape(1, n_loc)

    return pl.pallas_call(
        _normalize_body,
        grid=(grid,),
        in_specs=[
            pl.BlockSpec((BM, n_loc), lambda i: (i, 0)),
            pl.BlockSpec((BM, 1), lambda i: (i, 0)),
            pl.BlockSpec((1, n_loc), lambda i: (0, 0)),
        ],
        out_specs=pl.BlockSpec((BM, n_loc), lambda i: (i, 0)),
        out_shape=jax.ShapeDtypeStruct((m, n_loc), jnp.float32),
    )(x, inv, g2)


# device time: 72955 ns/iter; 1.0351x vs baseline; 1.0351x over previous
import jax
import jax.numpy as jnp
from jax import lax
from jax.experimental import pallas as pl
from jax.experimental.pallas import tpu as pltpu

N_DEV = 4
EPS = 1e-5
BM_SUMSQ = 2048
BM = 1024


def _sumsq_body(x_ref, out_ref):
    x = x_ref[...]
    out_ref[...] = jnp.sum(x * x, axis=1, keepdims=True)


def _allreduce_body(global_n, p_ref, inv_ref, comm_ref, send_sems, recv_sems):
    my = lax.axis_index("i")

    barrier = pltpu.get_barrier_semaphore()
    for j in range(1, N_DEV):
        pl.semaphore_signal(
            barrier, inc=1, device_id=(lax.rem(my + j, N_DEV),),
            device_id_type=pl.DeviceIdType.MESH,
        )
    pl.semaphore_wait(barrier, N_DEV - 1)

    comm_ref[0, :, :] = p_ref[...]

    rdmas = []
    for j in range(1, N_DEV):
        rdma = pltpu.make_async_remote_copy(
            src_ref=comm_ref.at[0],
            dst_ref=comm_ref.at[N_DEV - j],
            send_sem=send_sems.at[j - 1],
            recv_sem=recv_sems.at[N_DEV - 1 - j],
            device_id=(lax.rem(my + j, N_DEV),),
            device_id_type=pl.DeviceIdType.MESH,
        )
        rdma.start()
        rdmas.append(rdma)
    for rdma in rdmas:
        rdma.wait()

    total = (comm_ref[0, :, :] + comm_ref[1, :, :]
             + comm_ref[2, :, :] + comm_ref[3, :, :])
    inv_ref[...] = lax.rsqrt(total * (1.0 / global_n) + EPS)


def _normalize_body(x_ref, inv_ref, g_ref, out_ref):
    out_ref[...] = x_ref[...] * inv_ref[...] * g_ref[...]


def kernel(x, gamma):
    m, n_loc = x.shape
    global_n = n_loc * N_DEV
    partial = pl.pallas_call(
        _sumsq_body,
        grid=(m // BM_SUMSQ,),
        in_specs=[pl.BlockSpec((BM_SUMSQ, n_loc), lambda i: (i, 0))],
        out_specs=pl.BlockSpec((BM_SUMSQ, 1), lambda i: (i, 0)),
        out_shape=jax.ShapeDtypeStruct((m, 1), jnp.float32),
    )(x)

    p2 = partial.reshape(m // 128, 128)

    inv2 = pl.pallas_call(
        lambda *refs: _allreduce_body(global_n, *refs),
        out_shape=jax.ShapeDtypeStruct((m // 128, 128), jnp.float32),
        in_specs=[pl.BlockSpec(memory_space=pltpu.VMEM)],
        out_specs=pl.BlockSpec(memory_space=pltpu.VMEM),
        scratch_shapes=[
            pltpu.VMEM((N_DEV, m // 128, 128), jnp.float32),
            pltpu.SemaphoreType.DMA((N_DEV - 1,)),
            pltpu.SemaphoreType.DMA((N_DEV - 1,)),
        ],
        compiler_params=pltpu.CompilerParams(collective_id=0),
    )(p2)

    inv = inv2.reshape(m, 1)
    g2 = gamma.reshape(1, n_loc)

    return pl.pallas_call(
        _normalize_body,
        grid=(m // BM,),
        in_specs=[
            pl.BlockSpec((BM, n_loc), lambda i: (i, 0)),
            pl.BlockSpec((BM, 1), lambda i: (i, 0)),
            pl.BlockSpec((1, n_loc), lambda i: (0, 0)),
        ],
        out_specs=pl.BlockSpec((BM, n_loc), lambda i: (i, 0)),
        out_shape=jax.ShapeDtypeStruct((m, n_loc), jnp.float32),
    )(x, inv, g2)


# device time: 71141 ns/iter; 1.0615x vs baseline; 1.0255x over previous
import jax
import jax.numpy as jnp
from jax import lax
from jax.experimental import pallas as pl
from jax.experimental.pallas import tpu as pltpu

N_DEV = 4
EPS = 1e-5
BM = 1024


def _sumsq_allreduce_body(global_n, nblk, x_ref, inv_ref, comm_ref,
                          send_sems, recv_sems):
    i = pl.program_id(0)
    my = lax.axis_index("i")

    xb = x_ref[...]
    s = jnp.sum(xb * xb, axis=1, keepdims=True)

    r_idx = lax.broadcasted_iota(jnp.int32, (BM, 128), 0)
    l_idx = lax.broadcasted_iota(jnp.int32, (BM, 128), 1)
    w = jnp.where(r_idx % 128 == l_idx, s, 0.0)
    t_idx = lax.broadcasted_iota(jnp.int32, (BM // 128, BM), 0)
    rr_idx = lax.broadcasted_iota(jnp.int32, (BM // 128, BM), 1)
    m1 = jnp.where(rr_idx // 128 == t_idx, 1.0, 0.0).astype(jnp.float32)
    c = jax.lax.dot(m1, w, preferred_element_type=jnp.float32)

    rows = BM // 128
    comm_ref[0, pl.ds(i * rows, rows), :] = c

    @pl.when(i == nblk - 1)
    def _():
        barrier = pltpu.get_barrier_semaphore()
        for j in range(1, N_DEV):
            pl.semaphore_signal(
                barrier, inc=1, device_id=(lax.rem(my + j, N_DEV),),
                device_id_type=pl.DeviceIdType.MESH,
            )
        pl.semaphore_wait(barrier, N_DEV - 1)

        rdmas = []
        for j in range(1, N_DEV):
            rdma = pltpu.make_async_remote_copy(
                src_ref=comm_ref.at[0],
                dst_ref=comm_ref.at[N_DEV - j],
                send_sem=send_sems.at[j - 1],
                recv_sem=recv_sems.at[N_DEV - 1 - j],
                device_id=(lax.rem(my + j, N_DEV),),
                device_id_type=pl.DeviceIdType.MESH,
            )
            rdma.start()
            rdmas.append(rdma)
        for rdma in rdmas:
            rdma.wait()

        total = (comm_ref[0, :, :] + comm_ref[1, :, :]
                 + comm_ref[2, :, :] + comm_ref[3, :, :])
        inv_ref[...] = lax.rsqrt(total * (1.0 / global_n) + EPS)


def _normalize_body(x_ref, inv_ref, g_ref, out_ref):
    out_ref[...] = x_ref[...] * inv_ref[...] * g_ref[...]


def kernel(x, gamma):
    m, n_loc = x.shape
    global_n = n_loc * N_DEV
    nblk = m // BM

    inv2 = pl.pallas_call(
        lambda *refs: _sumsq_allreduce_body(global_n, nblk, *refs),
        grid=(nblk,),
        in_specs=[pl.BlockSpec((BM, n_loc), lambda i: (i, 0))],
        out_specs=pl.BlockSpec((m // 128, 128), lambda i: (0, 0)),
        out_shape=jax.ShapeDtypeStruct((m // 128, 128), jnp.float32),
        scratch_shapes=[
            pltpu.VMEM((N_DEV, m // 128, 128), jnp.float32),
            pltpu.SemaphoreType.DMA((N_DEV - 1,)),
            pltpu.SemaphoreType.DMA((N_DEV - 1,)),
        ],
        compiler_params=pltpu.CompilerParams(collective_id=0),
    )(x)

    inv = inv2.reshape(m, 1)
    g2 = gamma.reshape(1, n_loc)

    return pl.pallas_call(
        _normalize_body,
        grid=(nblk,),
        in_specs=[
            pl.BlockSpec((BM, n_loc), lambda i: (i, 0)),
            pl.BlockSpec((BM, 1), lambda i: (i, 0)),
            pl.BlockSpec((1, n_loc), lambda i: (0, 0)),
        ],
        out_specs=pl.BlockSpec((BM, n_loc), lambda i: (i, 0)),
        out_shape=jax.ShapeDtypeStruct((m, n_loc), jnp.float32),
    )(x, inv, g2)
